# baseline (speedup 1.0000x reference)
import jax
import jax.numpy as jnp
import numpy as np
from jax import lax
from jax.experimental import pallas as pl
from jax.experimental.pallas import tpu as pltpu
from jax.experimental.pallas import tpu_sc as plsc

N_LEVELS = 16
LOG2_T = 19
T = 1 << LOG2_T
TMASK = T - 1
BASE_RES = 16
PER_LEVEL_SCALE = 1.4472692012786865
N_PTS = 524288

RES = [int(np.floor(BASE_RES * PER_LEVEL_SCALE ** l)) for l in range(N_LEVELS)]
P1 = int(np.uint32(2654435761).astype(np.int64)) - (1 << 32)
P2 = 805459861


_MLP_BLK = 4096


def _mlp_body(x4_ref, w0_ref, w1_ref, w2_ref, o_ref):
    xb = x4_ref[...]
    x3 = xb[0:3, :]
    t1 = xb[3:4, :]
    ang_x = jnp.concatenate([x3 * (2.0 ** d) for d in range(4)], axis=0)
    ang_t = jnp.concatenate([t1 * (2.0 ** d) for d in range(4)], axis=0)
    h = jnp.concatenate(
        [jnp.sin(ang_x), jnp.cos(ang_x), jnp.sin(ang_t), jnp.cos(ang_t)], axis=0
    )
    dn = (((0,), (0,)), ((), ()))
    z0 = jnp.maximum(
        lax.dot_general(w0_ref[...], h, dn, preferred_element_type=jnp.float32), 0.0)
    z1 = jnp.maximum(
        lax.dot_general(w1_ref[...], z0, dn, preferred_element_type=jnp.float32), 0.0)
    dx = lax.dot_general(w2_ref[...], z1, dn, preferred_element_type=jnp.float32)
    o_ref[...] = x3 + dx


def _mlp_call(x4, W0, W1, W2):
    n = x4.shape[1]
    return pl.pallas_call(
        _mlp_body,
        grid=(n // _MLP_BLK,),
        in_specs=[
            pl.BlockSpec((4, _MLP_BLK), lambda i: (0, i)),
            pl.BlockSpec((32, 64), lambda i: (0, 0)),
            pl.BlockSpec((64, 64), lambda i: (0, 0)),
            pl.BlockSpec((64, 3), lambda i: (0, 0)),
        ],
        out_specs=pl.BlockSpec((3, _MLP_BLK), lambda i: (0, i)),
        out_shape=jax.ShapeDtypeStruct((3, n), jnp.float32),
    )(x4, W0, W1, W2)


NC = 2
NS = 16
NW = NC * NS
LANES = 16
PTS_PER_TILE = N_PTS // NW
C = 512
CHUNKS = PTS_PER_TILE // C
PAIRS = (CHUNKS - 2) // 2
RS = T // NS


def _sc_body(xdefT, tbl, out,
             sh0, sh1, xva, xvb, idxa, idxb, wva, wvb,
             r0a, r1a, r0b, r1b, oa, ob,
             sga, sgb, sx, soa, sob):
    cid = lax.axis_index("c")
    sid = lax.axis_index("s")
    wid = sid * NC + cid
    base0 = wid * PTS_PER_TILE

    def load_x(ch, xv):
        pltpu.async_copy(xdefT.at[:, pl.ds(base0 + ch * C, C)], xv, sx)

    def wait_x(ch, xv):
        pltpu.make_async_copy(
            xdefT.at[:, pl.ds(base0 + ch * C, C)], xv, sx).wait()

    def pass_a(res, xv, idxv, wv):
        def body(s, _):
            off = s * LANES
            x0 = xv[0, pl.ds(off, LANES)]
            x1 = xv[1, pl.ds(off, LANES)]
            x2 = xv[2, pl.ds(off, LANES)]

            def cellify(xj):
                pos = xj * res
                ci = pos.astype(jnp.int32)
                cf = ci.astype(jnp.float32)
                neg = cf > pos
                ci = jnp.where(neg, ci - 1, ci)
                cf = jnp.where(neg, cf - 1.0, cf)
                return ci, pos - cf

            c0, f0 = cellify(x0)
            c1, f1 = cellify(x1)
            c2, f2 = cellify(x2)
            m1 = c1 * P1
            m2 = c2 * P2
            m1b = m1 + P1
            m2b = m2 + P2
            c0b = c0 + 1
            a00 = lax.bitwise_xor(c0, m1)
            a01 = lax.bitwise_xor(c0b, m1)
            a10 = lax.bitwise_xor(c0, m1b)
            a11 = lax.bitwise_xor(c0b, m1b)
            g0 = 1.0 - f0
            g1 = 1.0 - f1
            g2 = 1.0 - f2
            w00 = g0 * g1
            w01 = f0 * g1
            w10 = g0 * f1
            w11 = f0 * f1
            corners = (
                (a00, m2, w00 * g2), (a01, m2, w01 * g2),
                (a10, m2, w10 * g2), (a11, m2, w11 * g2),
                (a00, m2b, w00 * f2), (a01, m2b, w01 * f2),
                (a10, m2b, w10 * f2), (a11, m2b, w11 * f2),
            )
            for c, (axy, mz, w) in enumerate(corners):
                idx = lax.bitwise_and(lax.bitwise_xor(axy, mz), TMASK)
                idxv[pl.ds(C * c + off, LANES)] = idx
                wv[c, pl.ds(off, LANES)] = w
            return 0

        lax.fori_loop(0, C // LANES, body, 0)

    def fire_gather(idxv, r0, r1, sem):
        pltpu.async_copy(sh0.at[idxv], r0, sem)
        pltpu.async_copy(sh1.at[idxv], r1, sem)

    def wait_gather(idxv, r0, r1, sem):
        pltpu.make_async_copy(sh0.at[idxv], r0, sem).wait()
        pltpu.make_async_copy(sh1.at[idxv], r1, sem).wait()

    def pass_b(wv, r0, r1, outv2):
        def body(s, _):
            off = s * LANES
            acc0 = jnp.zeros((LANES,), jnp.float32)
            acc1 = jnp.zeros((LANES,), jnp.float32)
            for c in range(8):
                w = wv[c, pl.ds(off, LANES)]
                v0 = r0[pl.ds(C * c + off, LANES)]
                v1 = r1[pl.ds(C * c + off, LANES)]
                acc0 = acc0 + w * v0
                acc1 = acc1 + w * v1
            outv2[0, pl.ds(off, LANES)] = acc0
            outv2[1, pl.ds(off, LANES)] = acc1
            return 0

        lax.fori_loop(0, C // LANES, body, 0)

    def out_dst(l, ch):
        return out.at[pl.ds(2 * l, 2), pl.ds(base0 + ch * C, C)]

    def fire_out(l, ch, outv2, sem):
        pltpu.async_copy(outv2, out_dst(l, ch), sem)

    def wait_out(l, ch, outv2, sem):
        pltpu.make_async_copy(outv2, out_dst(l, ch), sem).wait()

    def level_body(l, _):
        res = 0.0
        for i in range(N_LEVELS):
            res = jnp.where(l == i, float(RES[i]), res)
        pltpu.sync_copy(tbl.at[2 * l, pl.ds(sid * RS, RS)],
                        sh0.at[pl.ds(sid * RS, RS)])
        pltpu.sync_copy(tbl.at[2 * l + 1, pl.ds(sid * RS, RS)],
                        sh1.at[pl.ds(sid * RS, RS)])
        plsc.subcore_barrier()

        load_x(0, xva)
        wait_x(0, xva)
        load_x(1, xvb)
        pass_a(res, xva, idxa, wva)
        fire_gather(idxa, r0a, r1a, sga)

        def pair_body(k, _):
            chb = 2 * k + 1
            cha = 2 * k + 2
            wait_x(chb, xvb)
            pass_a(res, xvb, idxb, wvb)
            fire_gather(idxb, r0b, r1b, sgb)
            load_x(cha, xva)

            @pl.when(k > 0)
            def _():
                wait_out(l, cha - 4, oa, soa)
            wait_gather(idxa, r0a, r1a, sga)
            pass_b(wva, r0a, r1a, oa)
            fire_out(l, cha - 2, oa, soa)

            wait_x(cha, xva)
            pass_a(res, xva, idxa, wva)
            fire_gather(idxa, r0a, r1a, sga)
            load_x(cha + 1, xvb)

            @pl.when(k > 0)
            def _():
                wait_out(l, chb - 2, ob, sob)
            wait_gather(idxb, r0b, r1b, sgb)
            pass_b(wvb, r0b, r1b, ob)
            fire_out(l, chb, ob, sob)
            return 0

        lax.fori_loop(0, PAIRS, pair_body, 0)

        last = CHUNKS - 1
        wait_x(last, xvb)
        pass_a(res, xvb, idxb, wvb)
        fire_gather(idxb, r0b, r1b, sgb)

        wait_out(l, last - 3, oa, soa)
        wait_gather(idxa, r0a, r1a, sga)
        pass_b(wva, r0a, r1a, oa)
        fire_out(l, last - 1, oa, soa)

        wait_out(l, last - 2, ob, sob)
        wait_gather(idxb, r0b, r1b, sgb)
        pass_b(wvb, r0b, r1b, ob)
        fire_out(l, last, ob, sob)

        wait_out(l, last - 1, oa, soa)
        wait_out(l, last, ob, sob)
        plsc.subcore_barrier()
        return 0

    lax.fori_loop(0, N_LEVELS, level_body, 0)


def _sc_call(xdefT, tbl_planes):
    mesh = plsc.VectorSubcoreMesh(core_axis_name="c", subcore_axis_name="s")
    f = pl.kernel(
        _sc_body,
        out_type=jax.ShapeDtypeStruct((2 * N_LEVELS, N_PTS), jnp.float32),
        mesh=mesh,
        scratch_types=[
            pltpu.VMEM_SHARED((T,), jnp.float32),
            pltpu.VMEM_SHARED((T,), jnp.float32),
            pltpu.VMEM((3, C), jnp.float32),
            pltpu.VMEM((3, C), jnp.float32),
            pltpu.VMEM((8 * C,), jnp.int32),
            pltpu.VMEM((8 * C,), jnp.int32),
            pltpu.VMEM((8, C), jnp.float32),
            pltpu.VMEM((8, C), jnp.float32),
            pltpu.VMEM((8 * C,), jnp.float32),
            pltpu.VMEM((8 * C,), jnp.float32),
            pltpu.VMEM((8 * C,), jnp.float32),
            pltpu.VMEM((8 * C,), jnp.float32),
            pltpu.VMEM((2, C), jnp.float32),
            pltpu.VMEM((2, C), jnp.float32),
            pltpu.SemaphoreType.DMA,
            pltpu.SemaphoreType.DMA,
            pltpu.SemaphoreType.DMA,
            pltpu.SemaphoreType.DMA,
            pltpu.SemaphoreType.DMA,
        ],
    )
    return f(xdefT, tbl_planes)


def kernel(x, frame_time, table, W0, W1, W2):
    n = x.shape[0]
    xT = x.T
    t_row = jnp.broadcast_to(frame_time.reshape(1, 1), (1, n))
    x4 = jnp.concatenate([xT, t_row], axis=0)
    xdefT = _mlp_call(x4, W0, W1, W2)
    tbl_planes = table.transpose(0, 2, 1).reshape(2 * N_LEVELS, T)
    out32 = _sc_call(xdefT, tbl_planes)
    return out32.T


# speedup vs baseline: 1.0227x; 1.0227x over previous
import jax
import jax.numpy as jnp
import numpy as np
from jax import lax
from jax.experimental import pallas as pl
from jax.experimental.pallas import tpu as pltpu
from jax.experimental.pallas import tpu_sc as plsc

N_LEVELS = 16
LOG2_T = 19
T = 1 << LOG2_T
TMASK = T - 1
BASE_RES = 16
PER_LEVEL_SCALE = 1.4472692012786865
N_PTS = 524288

RES = [int(np.floor(BASE_RES * PER_LEVEL_SCALE ** l)) for l in range(N_LEVELS)]
P1 = int(np.uint32(2654435761).astype(np.int64)) - (1 << 32)
P2 = 805459861


_MLP_BLK = 4096


def _mlp_body(x4_ref, w0_ref, w1_ref, w2_ref, o_ref):
    xb = x4_ref[...]
    x3 = xb[0:3, :]
    t1 = xb[3:4, :]
    ang_x = jnp.concatenate([x3 * (2.0 ** d) for d in range(4)], axis=0)
    ang_t = jnp.concatenate([t1 * (2.0 ** d) for d in range(4)], axis=0)
    h = jnp.concatenate(
        [jnp.sin(ang_x), jnp.cos(ang_x), jnp.sin(ang_t), jnp.cos(ang_t)], axis=0
    )
    dn = (((0,), (0,)), ((), ()))
    z0 = jnp.maximum(
        lax.dot_general(w0_ref[...], h, dn, preferred_element_type=jnp.float32), 0.0)
    z1 = jnp.maximum(
        lax.dot_general(w1_ref[...], z0, dn, preferred_element_type=jnp.float32), 0.0)
    dx = lax.dot_general(w2_ref[...], z1, dn, preferred_element_type=jnp.float32)
    o_ref[...] = x3 + dx


def _mlp_call(x4, W0, W1, W2):
    n = x4.shape[1]
    return pl.pallas_call(
        _mlp_body,
        grid=(n // _MLP_BLK,),
        in_specs=[
            pl.BlockSpec((4, _MLP_BLK), lambda i: (0, i)),
            pl.BlockSpec((32, 64), lambda i: (0, 0)),
            pl.BlockSpec((64, 64), lambda i: (0, 0)),
            pl.BlockSpec((64, 3), lambda i: (0, 0)),
        ],
        out_specs=pl.BlockSpec((3, _MLP_BLK), lambda i: (0, i)),
        out_shape=jax.ShapeDtypeStruct((3, n), jnp.float32),
    )(x4, W0, W1, W2)


NC = 2
NS = 16
NW = NC * NS
LANES = 16
PTS_PER_TILE = N_PTS // NW
C = 512
CHUNKS = PTS_PER_TILE // C
PAIRS = (CHUNKS - 2) // 2
RS = T // NS


def _sc_body(xdefT, tbl, out,
             sh0, sh1, xva, xvb, idxa, idxb, wva, wvb,
             r0a, r1a, r0b, r1b, oa, ob,
             sga, sgb, sx, soa, sob):
    cid = lax.axis_index("c")
    sid = lax.axis_index("s")
    wid = sid * NC + cid
    base0 = wid * PTS_PER_TILE

    def load_x(ch, xv):
        pltpu.async_copy(xdefT.at[:, pl.ds(base0 + ch * C, C)], xv, sx)

    def wait_x(ch, xv):
        pltpu.make_async_copy(
            xdefT.at[:, pl.ds(base0 + ch * C, C)], xv, sx).wait()

    def pass_a(res, xv, idxv, wv):
        @plsc.parallel_loop(0, C // LANES, unroll=2)
        def body(s):
            off = s * LANES
            x0 = xv[0, pl.ds(off, LANES)]
            x1 = xv[1, pl.ds(off, LANES)]
            x2 = xv[2, pl.ds(off, LANES)]

            def cellify(xj):
                pos = xj * res
                ci = pos.astype(jnp.int32)
                cf = ci.astype(jnp.float32)
                neg = cf > pos
                ci = jnp.where(neg, ci - 1, ci)
                cf = jnp.where(neg, cf - 1.0, cf)
                return ci, pos - cf

            c0, f0 = cellify(x0)
            c1, f1 = cellify(x1)
            c2, f2 = cellify(x2)
            m1 = c1 * P1
            m2 = c2 * P2
            m1b = m1 + P1
            m2b = m2 + P2
            c0b = c0 + 1
            a00 = lax.bitwise_xor(c0, m1)
            a01 = lax.bitwise_xor(c0b, m1)
            a10 = lax.bitwise_xor(c0, m1b)
            a11 = lax.bitwise_xor(c0b, m1b)
            g0 = 1.0 - f0
            g1 = 1.0 - f1
            g2 = 1.0 - f2
            w00 = g0 * g1
            w01 = f0 * g1
            w10 = g0 * f1
            w11 = f0 * f1
            corners = (
                (a00, m2, w00 * g2), (a01, m2, w01 * g2),
                (a10, m2, w10 * g2), (a11, m2, w11 * g2),
                (a00, m2b, w00 * f2), (a01, m2b, w01 * f2),
                (a10, m2b, w10 * f2), (a11, m2b, w11 * f2),
            )
            for c, (axy, mz, w) in enumerate(corners):
                idx = lax.bitwise_and(lax.bitwise_xor(axy, mz), TMASK)
                idxv[pl.ds(C * c + off, LANES)] = idx
                wv[c, pl.ds(off, LANES)] = w

    def fire_gather(idxv, r0, r1, sem):
        pltpu.async_copy(sh0.at[idxv], r0, sem)
        pltpu.async_copy(sh1.at[idxv], r1, sem)

    def wait_gather(idxv, r0, r1, sem):
        pltpu.make_async_copy(sh0.at[idxv], r0, sem).wait()
        pltpu.make_async_copy(sh1.at[idxv], r1, sem).wait()

    def pass_b(wv, r0, r1, outv2):
        @plsc.parallel_loop(0, C // LANES, unroll=2)
        def body(s):
            off = s * LANES
            acc0 = jnp.zeros((LANES,), jnp.float32)
            acc1 = jnp.zeros((LANES,), jnp.float32)
            for c in range(8):
                w = wv[c, pl.ds(off, LANES)]
                v0 = r0[pl.ds(C * c + off, LANES)]
                v1 = r1[pl.ds(C * c + off, LANES)]
                acc0 = acc0 + w * v0
                acc1 = acc1 + w * v1
            outv2[0, pl.ds(off, LANES)] = acc0
            outv2[1, pl.ds(off, LANES)] = acc1

    def out_dst(l, ch):
        return out.at[pl.ds(2 * l, 2), pl.ds(base0 + ch * C, C)]

    def fire_out(l, ch, outv2, sem):
        pltpu.async_copy(outv2, out_dst(l, ch), sem)

    def wait_out(l, ch, outv2, sem):
        pltpu.make_async_copy(outv2, out_dst(l, ch), sem).wait()

    def level_body(l, _):
        res = 0.0
        for i in range(N_LEVELS):
            res = jnp.where(l == i, float(RES[i]), res)
        pltpu.sync_copy(tbl.at[2 * l, pl.ds(sid * RS, RS)],
                        sh0.at[pl.ds(sid * RS, RS)])
        pltpu.sync_copy(tbl.at[2 * l + 1, pl.ds(sid * RS, RS)],
                        sh1.at[pl.ds(sid * RS, RS)])
        plsc.subcore_barrier()

        load_x(0, xva)
        wait_x(0, xva)
        load_x(1, xvb)
        pass_a(res, xva, idxa, wva)
        fire_gather(idxa, r0a, r1a, sga)

        def pair_body(k, _):
            chb = 2 * k + 1
            cha = 2 * k + 2
            wait_x(chb, xvb)
            pass_a(res, xvb, idxb, wvb)
            fire_gather(idxb, r0b, r1b, sgb)
            load_x(cha, xva)

            @pl.when(k > 0)
            def _():
                wait_out(l, cha - 4, oa, soa)
            wait_gather(idxa, r0a, r1a, sga)
            pass_b(wva, r0a, r1a, oa)
            fire_out(l, cha - 2, oa, soa)

            wait_x(cha, xva)
            pass_a(res, xva, idxa, wva)
            fire_gather(idxa, r0a, r1a, sga)
            load_x(cha + 1, xvb)

            @pl.when(k > 0)
            def _():
                wait_out(l, chb - 2, ob, sob)
            wait_gather(idxb, r0b, r1b, sgb)
            pass_b(wvb, r0b, r1b, ob)
            fire_out(l, chb, ob, sob)
            return 0

        lax.fori_loop(0, PAIRS, pair_body, 0)

        last = CHUNKS - 1
        wait_x(last, xvb)
        pass_a(res, xvb, idxb, wvb)
        fire_gather(idxb, r0b, r1b, sgb)

        wait_out(l, last - 3, oa, soa)
        wait_gather(idxa, r0a, r1a, sga)
        pass_b(wva, r0a, r1a, oa)
        fire_out(l, last - 1, oa, soa)

        wait_out(l, last - 2, ob, sob)
        wait_gather(idxb, r0b, r1b, sgb)
        pass_b(wvb, r0b, r1b, ob)
        fire_out(l, last, ob, sob)

        wait_out(l, last - 1, oa, soa)
        wait_out(l, last, ob, sob)
        plsc.subcore_barrier()
        return 0

    lax.fori_loop(0, N_LEVELS, level_body, 0)


def _sc_call(xdefT, tbl_planes):
    mesh = plsc.VectorSubcoreMesh(core_axis_name="c", subcore_axis_name="s")
    f = pl.kernel(
        _sc_body,
        out_type=jax.ShapeDtypeStruct((2 * N_LEVELS, N_PTS), jnp.float32),
        mesh=mesh,
        scratch_types=[
            pltpu.VMEM_SHARED((T,), jnp.float32),
            pltpu.VMEM_SHARED((T,), jnp.float32),
            pltpu.VMEM((3, C), jnp.float32),
            pltpu.VMEM((3, C), jnp.float32),
            pltpu.VMEM((8 * C,), jnp.int32),
            pltpu.VMEM((8 * C,), jnp.int32),
            pltpu.VMEM((8, C), jnp.float32),
            pltpu.VMEM((8, C), jnp.float32),
            pltpu.VMEM((8 * C,), jnp.float32),
            pltpu.VMEM((8 * C,), jnp.float32),
            pltpu.VMEM((8 * C,), jnp.float32),
            pltpu.VMEM((8 * C,), jnp.float32),
            pltpu.VMEM((2, C), jnp.float32),
            pltpu.VMEM((2, C), jnp.float32),
            pltpu.SemaphoreType.DMA,
            pltpu.SemaphoreType.DMA,
            pltpu.SemaphoreType.DMA,
            pltpu.SemaphoreType.DMA,
            pltpu.SemaphoreType.DMA,
        ],
    )
    return f(xdefT, tbl_planes)


def kernel(x, frame_time, table, W0, W1, W2):
    n = x.shape[0]
    xT = x.T
    t_row = jnp.broadcast_to(frame_time.reshape(1, 1), (1, n))
    x4 = jnp.concatenate([xT, t_row], axis=0)
    xdefT = _mlp_call(x4, W0, W1, W2)
    tbl_planes = table.transpose(0, 2, 1).reshape(2 * N_LEVELS, T)
    out32 = _sc_call(xdefT, tbl_planes)
    return out32.T
